# baseline (speedup 1.0000x reference)
import functools

import jax
import jax.numpy as jnp
from jax import lax
from jax.experimental import pallas as pl
from jax.experimental.pallas import tpu as pltpu
from jax.experimental.pallas import tpu_sc as plsc

_NC = 2
_NS = 16
_NW = _NC * _NS


def _make(N, S, V, D):
    NPW = N // _NW
    KPW = NPW * S
    RC = 1024
    NCH = 256
    mesh = plsc.VectorSubcoreMesh(core_axis_name="c", subcore_axis_name="s")

    @functools.partial(
        pl.kernel,
        mesh=mesh,
        out_type=jax.ShapeDtypeStruct((S, D, N), jnp.float32),
        scratch_types=[
            pltpu.VMEM((RC,), jnp.int32),
            pltpu.VMEM((S * NPW,), jnp.int32),
            pltpu.VMEM((NCH, 2 * D), jnp.float32),
            pltpu.VMEM((NCH, 2 * D), jnp.float32),
            pltpu.VMEM((D, NCH), jnp.float32),
            pltpu.VMEM((D, NCH), jnp.float32),
            pltpu.SemaphoreType.DMA,
            pltpu.SemaphoreType.DMA,
            pltpu.SemaphoreType.DMA,
            pltpu.SemaphoreType.DMA,
            pltpu.SemaphoreType.DMA,
        ],
        compiler_params=pltpu.CompilerParams(
            use_tc_tiling_on_sc=True, needs_layout_passes=False),
    )
    def k(idx_hbm, table_hbm, out_hbm, idx_v, idx_s, b0, b1, w0, w1,
          isem, g0, g1, s0, s1):
        bufs = (b0, b1)
        wbufs = (w0, w1)
        gsems = (g0, g1)
        wsems = (s0, s1)
        wid = lax.axis_index("s") * _NC + lax.axis_index("c")
        kbase = wid * KPW
        nbase = wid * NPW
        lane = lax.iota(jnp.int32, 16)

        def stage_chunk(c, carry):
            pltpu.sync_copy(idx_hbm.at[pl.ds(kbase + c * RC, RC)], idx_v)

            def scat(kb, carry2):
                kl = c * RC + kb * 16
                vals = idx_v[pl.ds(kb * 16, 16)]
                kvec = kl + lane
                svec = lax.rem(kvec, S)
                nvec = lax.div(kvec, S)
                plsc.store_scatter(idx_s, [svec * NPW + nvec], vals)
                return carry2

            lax.fori_loop(0, RC // 16, scat, 0)
            return carry

        lax.fori_loop(0, KPW // RC, stage_chunk, 0)

        n_iters = S * (NPW // NCH)

        def gather(it, b):
            s = it // (NPW // NCH)
            h = it % (NPW // NCH)
            pltpu.async_copy(
                table_hbm.at[idx_s.at[pl.ds(s * NPW + h * NCH, NCH)]],
                bufs[b], gsems[b])

        def wait_gather(it, b):
            s = it // (NPW // NCH)
            h = it % (NPW // NCH)
            pltpu.make_async_copy(
                table_hbm.at[idx_s.at[pl.ds(s * NPW + h * NCH, NCH)]],
                bufs[b], gsems[b]).wait()

        def write(it, b):
            s = it // (NPW // NCH)
            h = it % (NPW // NCH)
            pltpu.async_copy(
                wbufs[b], out_hbm.at[s, :, pl.ds(nbase + h * NCH, NCH)],
                wsems[b])

        def wait_write(it, b):
            s = it // (NPW // NCH)
            h = it % (NPW // NCH)
            pltpu.make_async_copy(
                wbufs[b], out_hbm.at[s, :, pl.ds(nbase + h * NCH, NCH)],
                wsems[b]).wait()

        dvecs = [jnp.int32(d0) + lane for d0 in range(0, D, 16)]

        def transpose(b):
            buf = bufs[b]
            wbuf = wbufs[b]

            def per_m(m, carry):
                mvec = jnp.full((16,), m, jnp.int32)
                for i, d0 in enumerate(range(0, D, 16)):
                    v = buf[m, pl.ds(d0, 16)]
                    plsc.store_scatter(wbuf, [dvecs[i], mvec], v)
                return carry

            lax.fori_loop(0, NCH, per_m, 0)

        gather(0, 0)

        def body(g, carry):
            it0 = g * 2
            for b in range(2):
                it = it0 + b

                @pl.when(it + 1 < n_iters)
                def _(it=it, b=b):
                    gather(it + 1, 1 - b)

                wait_gather(it, b)

                @pl.when(it >= 2)
                def _(it=it, b=b):
                    wait_write(it - 2, b)

                transpose(b)
                write(it, b)
            return carry

        lax.fori_loop(0, n_iters // 2, body, 0)
        wait_write(n_iters - 2, 0)
        wait_write(n_iters - 1, 1)

    return k


def kernel(input, table):
    N, S = input.shape
    V, D = table.shape
    idxf = input.reshape(-1)
    tablep = jnp.pad(table, ((0, 0), (0, 128 - D)))
    out_t = _make(N, S, V, D)(idxf, tablep)
    return jnp.transpose(out_t, (2, 0, 1))


# speedup vs baseline: 1.0016x; 1.0016x over previous
import functools

import jax
import jax.numpy as jnp
from jax import lax
from jax.experimental import pallas as pl
from jax.experimental.pallas import tpu as pltpu
from jax.experimental.pallas import tpu_sc as plsc

_NC = 2
_NS = 16
_NW = _NC * _NS


def _make(N, S, V, D):
    NPW = N // _NW
    KPW = NPW * S
    RC = 1024
    NCH = 256
    mesh = plsc.VectorSubcoreMesh(core_axis_name="c", subcore_axis_name="s")

    @functools.partial(
        pl.kernel,
        mesh=mesh,
        out_type=jax.ShapeDtypeStruct((S, D, N), jnp.float32),
        scratch_types=[
            pltpu.VMEM((RC,), jnp.int32),
            pltpu.VMEM((S * NPW,), jnp.int32),
            pltpu.VMEM((NCH, 2 * D), jnp.float32),
            pltpu.VMEM((NCH, 2 * D), jnp.float32),
            pltpu.VMEM((D, NCH), jnp.float32),
            pltpu.VMEM((D, NCH), jnp.float32),
            pltpu.SemaphoreType.DMA,
            pltpu.SemaphoreType.DMA,
            pltpu.SemaphoreType.DMA,
            pltpu.SemaphoreType.DMA,
            pltpu.SemaphoreType.DMA,
        ],
        compiler_params=pltpu.CompilerParams(
            use_tc_tiling_on_sc=True, needs_layout_passes=False),
    )
    def k(idx_hbm, table_hbm, out_hbm, idx_v, idx_s, b0, b1, w0, w1,
          isem, g0, g1, s0, s1):
        bufs = (b0, b1)
        wbufs = (w0, w1)
        gsems = (g0, g1)
        wsems = (s0, s1)
        wid = lax.axis_index("s") * _NC + lax.axis_index("c")
        kbase = wid * KPW
        nbase = wid * NPW
        lane = lax.iota(jnp.int32, 16)

        def stage_chunk(c, carry):
            pltpu.sync_copy(idx_hbm.at[pl.ds(kbase + c * RC, RC)], idx_v)

            def scat(kb, carry2):
                kl = c * RC + kb * 16
                vals = idx_v[pl.ds(kb * 16, 16)]
                kvec = kl + lane
                svec = lax.rem(kvec, S)
                nvec = lax.div(kvec, S)
                plsc.store_scatter(idx_s, [svec * NPW + nvec], vals)
                return carry2

            lax.fori_loop(0, RC // 16, scat, 0)
            return carry

        lax.fori_loop(0, KPW // RC, stage_chunk, 0)

        n_iters = S * (NPW // NCH)

        def gather(it, b):
            s = it // (NPW // NCH)
            h = it % (NPW // NCH)
            pltpu.async_copy(
                table_hbm.at[idx_s.at[pl.ds(s * NPW + h * NCH, NCH)]],
                bufs[b], gsems[b])

        def wait_gather(it, b):
            s = it // (NPW // NCH)
            h = it % (NPW // NCH)
            pltpu.make_async_copy(
                table_hbm.at[idx_s.at[pl.ds(s * NPW + h * NCH, NCH)]],
                bufs[b], gsems[b]).wait()

        def write(it, b):
            s = it // (NPW // NCH)
            h = it % (NPW // NCH)
            pltpu.async_copy(
                wbufs[b], out_hbm.at[s, :, pl.ds(nbase + h * NCH, NCH)],
                wsems[b])

        def wait_write(it, b):
            s = it // (NPW // NCH)
            h = it % (NPW // NCH)
            pltpu.make_async_copy(
                wbufs[b], out_hbm.at[s, :, pl.ds(nbase + h * NCH, NCH)],
                wsems[b]).wait()

        dvecs = [jnp.int32(d0) + lane for d0 in range(0, D, 16)]

        def transpose(b):
            buf = bufs[b]
            wbuf = wbufs[b]

            def per_m(g, mvec):
                for u in range(8):
                    m = g * 8 + u
                    mv = mvec + u
                    for i, d0 in enumerate(range(0, D, 16)):
                        v = buf[m, pl.ds(d0, 16)]
                        plsc.store_scatter(wbuf, [dvecs[i], mv], v)
                return mvec + 8

            lax.fori_loop(0, NCH // 8, per_m, jnp.zeros((16,), jnp.int32))

        gather(0, 0)

        def body(g, carry):
            it0 = g * 2
            for b in range(2):
                it = it0 + b

                @pl.when(it + 1 < n_iters)
                def _(it=it, b=b):
                    gather(it + 1, 1 - b)

                wait_gather(it, b)

                @pl.when(it >= 2)
                def _(it=it, b=b):
                    wait_write(it - 2, b)

                transpose(b)
                write(it, b)
            return carry

        lax.fori_loop(0, n_iters // 2, body, 0)
        wait_write(n_iters - 2, 0)
        wait_write(n_iters - 1, 1)

    return k


def kernel(input, table):
    N, S = input.shape
    V, D = table.shape
    idxf = input.reshape(-1)
    tablep = jnp.pad(table, ((0, 0), (0, 128 - D)))
    out_t = _make(N, S, V, D)(idxf, tablep)
    return jnp.transpose(out_t, (2, 0, 1))
